# baseline (speedup 1.0000x reference)
import jax
import jax.numpy as jnp
import numpy as np
from jax.experimental import pallas as pl
from jax.experimental.pallas import tpu as pltpu

_N = 20000
_G = 100
_IMG = 1344.0
_ROWS = 160
_LANES = 128
_NP = _ROWS * _LANES
_K = 128


def _make_ranks():
    kp, kn = jax.random.split(jax.random.key(42))
    out = []
    for k in (kp, kn):
        pri = np.asarray(jax.random.uniform(k, (_N,)))
        perm = np.argsort(-pri, kind="stable")
        rank = np.empty(_N, np.int32)
        rank[perm] = np.arange(_N, dtype=np.int32)
        pad = np.full(_NP - _N, np.int32(1 << 30), np.int32)
        out.append(np.concatenate([rank, pad]).reshape(_ROWS, _LANES))
    return out[0], out[1]


_RANK_POS, _RANK_NEG = _make_ranks()


def _body(gt_ref, a_ref, v_ref, rp_ref, rn_ref,
          lab_ref, lw_ref, posf_ref, tgt_ref, npos_ref, nneg_ref):
    ax1 = a_ref[0]
    ay1 = a_ref[1]
    ax2 = a_ref[2]
    ay2 = a_ref[3]

    a1 = (ax2 - ax1 + 1.0) * (ay2 - ay1 + 1.0)
    mo = jnp.full((_ROWS, _LANES), -jnp.inf, jnp.float32)
    am = jnp.zeros((_ROWS, _LANES), jnp.int32)
    lq = jnp.full((_ROWS, _LANES), -1, jnp.int32)
    for g in range(_G):
        gx1 = gt_ref[0, g]
        gy1 = gt_ref[1, g]
        gx2 = gt_ref[2, g]
        gy2 = gt_ref[3, g]
        a2 = (gx2 - gx1 + 1.0) * (gy2 - gy1 + 1.0)
        wx = jnp.maximum(
            jnp.minimum(ax2, gx2) - jnp.maximum(ax1, gx1) + 1.0, 0.0)
        wy = jnp.maximum(
            jnp.minimum(ay2, gy2) - jnp.maximum(ay1, gy1) + 1.0, 0.0)
        inter = wx * wy
        iou = inter / (a1 + a2 - inter)
        gmax = jnp.max(iou)
        lqt = jnp.where(gmax >= 0.3, gmax - 1e-6, jnp.float32(jnp.inf))
        better = iou > mo
        mo = jnp.where(better, iou, mo)
        am = jnp.where(better, g, am)
        lq = jnp.where(iou >= lqt, g, lq)

    inside = ((v_ref[...] != 0) & (ax1 >= 0.0) & (ay1 >= 0.0)
              & (ax2 < _IMG) & (ay2 < _IMG))
    has_lq = lq >= 0
    pos_m = inside & ((mo >= 0.7) | has_lq)
    neg_m = inside & (mo >= -1.0) & (mo < 0.3) & (~has_lq)

    rp = rp_ref[...]
    rn = rn_ref[...]

    def round16(mask, rank, lo, s, njs):
        cs = [jnp.sum(jnp.where(mask & (rank <= lo + (j * s - 1)), 1, 0))
              for j in range(njs)]
        m = cs[0] * 0
        for c in cs:
            m = m + jnp.where(c < _K, 1, 0)
        return lo + (m - 1) * s

    tp = jnp.int32(0)
    tn = jnp.int32(0)
    for s, njs in ((2048, 16), (128, 16), (8, 16), (1, 8)):
        tp = round16(pos_m, rp, tp, s, njs)
        tn = round16(neg_m, rn, tn, s, njs)
    sp = pos_m & (rp <= tp)
    sn = neg_m & (rn <= tn)

    lab_ref[...] = jnp.where(sp, 1, 0)
    lw_ref[...] = jnp.where(sp | sn, 1.0, 0.0)
    posf_ref[...] = jnp.where(sp, 1.0, 0.0)
    npos_ref[0, 0] = jnp.sum(jnp.where(sp, 1, 0))
    nneg_ref[0, 0] = jnp.sum(jnp.where(sn, 1, 0))

    gidx = jnp.where(has_lq, lq, am)

    zf = jnp.zeros((_ROWS, _LANES), jnp.float32)
    mx1, my1, mx2, my2 = zf, zf, zf, zf
    for j in range(_G):
        m = gidx == j
        mx1 = jnp.where(m, gt_ref[0, j], mx1)
        my1 = jnp.where(m, gt_ref[1, j], my1)
        mx2 = jnp.where(m, gt_ref[2, j], mx2)
        my2 = jnp.where(m, gt_ref[3, j], my2)
    px = (ax1 + ax2) * 0.5
    py = (ay1 + ay2) * 0.5
    pw = ax2 - ax1 + 1.0
    ph = ay2 - ay1 + 1.0
    gx = (mx1 + mx2) * 0.5
    gy = (my1 + my2) * 0.5
    gw = mx2 - mx1 + 1.0
    gh = my2 - my1 + 1.0
    tgt_ref[0] = jnp.where(sp, (gx - px) / pw, 0.0)
    tgt_ref[1] = jnp.where(sp, (gy - py) / ph, 0.0)
    tgt_ref[2] = jnp.where(sp, jnp.log(gw / pw), 0.0)
    tgt_ref[3] = jnp.where(sp, jnp.log(gh / ph), 0.0)


def _run(a4, v2, gt4, rp, rn):
    f32 = jnp.float32
    i32 = jnp.int32
    vmem2 = pl.BlockSpec((_ROWS, _LANES), lambda: (0, 0))
    return pl.pallas_call(
        _body,
        in_specs=[
            pl.BlockSpec(memory_space=pltpu.SMEM),
            pl.BlockSpec((4, _ROWS, _LANES), lambda: (0, 0, 0)),
            vmem2,
            vmem2,
            vmem2,
        ],
        out_specs=[
            vmem2,
            vmem2,
            vmem2,
            pl.BlockSpec((4, _ROWS, _LANES), lambda: (0, 0, 0)),
            pl.BlockSpec(memory_space=pltpu.SMEM),
            pl.BlockSpec(memory_space=pltpu.SMEM),
        ],
        out_shape=[
            jax.ShapeDtypeStruct((_ROWS, _LANES), i32),
            jax.ShapeDtypeStruct((_ROWS, _LANES), f32),
            jax.ShapeDtypeStruct((_ROWS, _LANES), f32),
            jax.ShapeDtypeStruct((4, _ROWS, _LANES), f32),
            jax.ShapeDtypeStruct((1, 1), i32),
            jax.ShapeDtypeStruct((1, 1), i32),
        ],
    )(gt4, a4, v2, rp, rn)


def kernel(anchors, valid_flags, gt_bboxes):
    pad_box = jnp.array([-1e6, -1e6, -1e6 + 100.0, -1e6 + 100.0], jnp.float32)
    a_p = jnp.concatenate(
        [anchors, jnp.broadcast_to(pad_box, (_NP - _N, 4))], axis=0)
    a4 = a_p.T.reshape(4, _ROWS, _LANES)
    v2 = jnp.concatenate(
        [valid_flags.astype(jnp.int32),
         jnp.zeros((_NP - _N,), jnp.int32)]).reshape(_ROWS, _LANES)
    gt4 = gt_bboxes.T
    rp = jnp.asarray(_RANK_POS)
    rn = jnp.asarray(_RANK_NEG)

    lab, lw, posf, tgt, npos, nneg = _run(a4, v2, gt4, rp, rn)

    labels = lab.reshape(-1)[:_N]
    label_weights = lw.reshape(-1)[:_N]
    bbox_targets = tgt.reshape(4, -1)[:, :_N].T
    posf1 = posf.reshape(-1)[:_N]
    bbox_weights = jnp.broadcast_to(posf1[:, None], (_N, 4))
    num_pos = npos[0, 0]
    num_neg = nneg[0, 0]
    return labels, label_weights, bbox_targets, bbox_weights, num_pos, num_neg
